# Initial kernel scaffold; baseline (speedup 1.0000x reference)
import jax
import jax.numpy as jnp
from jax.experimental import pallas as pl


def kernel(x, pos, query_pos, qmw, qmo, W_lin, b_lin, W_mlp1, b_mlp1, W_mlp2, b_mlp2, B_rff, W_l1, b_l1, W_l2, filt, bias):
    raise NotImplementedError("write your pallas kernel here")


# speedup vs baseline: 1.3694x; 1.3694x over previous
import functools

import jax
import jax.numpy as jnp
import numpy as np
from jax.experimental import pallas as pl

_IN_C = 32
_OUT_C = 64
_NQ = 256
_V = 2048
_B = 8
_K = 103
_KP = 104
_QB = 8
_EPS = 1e-8
_SQRT1_2 = 0.7071067811865476


def _gelu(v):
    return 0.5 * v * (1.0 + jax.lax.erf(v * _SQRT1_2))


def _xlin_kernel(x_ref, w_ref, b_ref, o_ref):
    o_ref[...] = (
        jnp.dot(x_ref[...], w_ref[...], preferred_element_type=jnp.float32)
        + b_ref[...]
    )


def _mega_kernel(xl2_ref, posT_ref, pos8_ref, qp_ref, qmw_ref, qmo_ref,
                 brff_ref, wl1_ref, bl1_ref, wl2big_ref, bias_ref,
                 w1_ref, bm1_ref, w2_ref, bm2_ref, out_ref):
    f32 = jnp.float32
    qp = qp_ref[...]
    p0 = posT_ref[0:1, :]
    p1 = posT_ref[1:2, :]
    d0 = qp[:, 0:1] - p0
    d0 = d0 + 0.5
    d0 = d0 - jnp.floor(d0) - 0.5
    d1 = qp[:, 1:2] - p1
    d1 = d1 + 0.5
    d1 = d1 - jnp.floor(d1) - 0.5
    ed = d0 * d0 + d1 * d1

    bits = jax.lax.bitcast_convert_type(ed, jnp.int32)
    prefix = jnp.zeros((_QB, 1), jnp.int32)
    krem = jnp.full((_QB, 1), _K, jnp.int32)
    for bit in range(30, -1, -1):
        high = jax.lax.shift_right_logical(bits, bit + 1)
        bitv = jax.lax.shift_right_logical(bits, bit) & 1
        match0 = jnp.logical_and(high == prefix, bitv == 0)
        cnt0 = jnp.sum(match0.astype(jnp.int32), axis=1, keepdims=True)
        take0 = krem <= cnt0
        prefix = (prefix << 1) | jnp.where(take0, 0, 1)
        krem = jnp.where(take0, krem, krem - cnt0)
    thr_bits = prefix
    sel = bits <= thr_bits
    seli = sel.astype(jnp.int32)
    cs = seli
    sh = 1
    while sh < _V:
        cs = cs + jnp.concatenate(
            [jnp.zeros((_QB, sh), jnp.int32), cs[:, :-sh]], axis=1)
        sh *= 2
    sel2 = jnp.logical_and(sel, cs <= _K)
    posn = cs - 1

    thr_val = jax.lax.bitcast_convert_type(thr_bits, f32)
    mn = jnp.min(ed, axis=1, keepdims=True)
    en = (ed - mn) / (thr_val - mn + _EPS)
    w = jnp.exp(-en) * sel2.astype(f32)
    kd = w / jnp.sum(w, axis=1, keepdims=True)

    iota = jax.lax.broadcasted_iota(jnp.int32, (_KP, 1), 0)
    xl2 = xl2_ref[...]
    wl2big = wl2big_ref[...]

    ys = []
    for i in range(_QB):
        onehot = jnp.logical_and(posn[i:i + 1, :] == iota,
                                 sel2[i:i + 1, :]).astype(f32)
        at_kd = onehot * kd[i:i + 1, :]
        gxk = jax.lax.dot_general(
            at_kd, xl2, (((1,), (0,)), ((), ())),
            preferred_element_type=f32)
        pg = jnp.dot(onehot, pos8_ref[...],
                     preferred_element_type=f32)
        dg0 = qp[i:i + 1, 0:1] - pg[:, 0:1]
        dg0 = dg0 + 0.5
        dg0 = dg0 - jnp.floor(dg0) - 0.5
        dg1 = qp[i:i + 1, 1:2] - pg[:, 1:2]
        dg1 = dg1 + 0.5
        dg1 = dg1 - jnp.floor(dg1) - 0.5
        proj = (2.0 * np.pi) * (dg0 * brff_ref[0:1, :] + dg1 * brff_ref[1:2, :])
        kf = jnp.concatenate([jnp.sin(proj), jnp.cos(proj)], axis=1)
        h = jnp.dot(kf, wl1_ref[...], preferred_element_type=f32) + bl1_ref[...]
        h = h * qmw_ref[i:i + 1, :] + qmo_ref[i:i + 1, :]
        h = _gelu(h)
        ha = jnp.concatenate(
            [h, jnp.ones((_KP, 1), f32), jnp.zeros((_KP, 7), f32)], axis=1)
        mt = jax.lax.dot_general(
            gxk, ha, (((0,), (0,)), ((), ())),
            preferred_element_type=f32)
        a = jnp.concatenate(
            [mt[c * 8:(c + 1) * 8, :] for c in range(_IN_C)],
            axis=1)
        ys.append(jnp.dot(a, wl2big, preferred_element_type=f32))
    y = jnp.stack(ys, axis=1)
    z = _gelu(y + bias_ref[...].reshape(1, 1, _OUT_C))
    zr = z.reshape(_B * _QB, _OUT_C)
    m = _gelu(jnp.dot(zr, w1_ref[...], preferred_element_type=f32)
              + bm1_ref[...])
    m = jnp.dot(m, w2_ref[...], preferred_element_type=f32) + bm2_ref[...]
    out_ref[...] = (m + zr).reshape(_B, _QB, _OUT_C)


@functools.partial(jax.jit, static_argnames=())
def _run(x, pos, query_pos, qmw, qmo, W_lin, b_lin, W_mlp1, b_mlp1,
         W_mlp2, b_mlp2, B_rff, W_l1, b_l1, W_l2, filt, bias):
    f32 = jnp.float32
    xf = x.reshape(_B * _V, _IN_C)
    xl = pl.pallas_call(
        _xlin_kernel,
        out_shape=jax.ShapeDtypeStruct((_B * _V, _IN_C), f32),
    )(xf, W_lin.T, b_lin.reshape(1, _IN_C))
    xl2 = xl.reshape(_B, _V, _IN_C).transpose(1, 2, 0).reshape(_V, _IN_C * _B)
    posT = pos.T
    pos8 = jnp.concatenate([pos, jnp.zeros((_V, 6), f32)], axis=1)
    wa = jnp.concatenate(
        [W_l2.T, filt.reshape(1, _V), jnp.zeros((7, _V), f32)], axis=0)
    wl2big = wa.reshape(40, _IN_C, _OUT_C).transpose(1, 0, 2).reshape(
        _IN_C * 40, _OUT_C)

    grid = (_NQ // _QB,)
    full = lambda shape: pl.BlockSpec(shape, lambda i: tuple(0 for _ in shape))
    out = pl.pallas_call(
        _mega_kernel,
        grid=grid,
        in_specs=[
            full((_V, _IN_C * _B)),
            full((2, _V)),
            full((_V, 8)),
            pl.BlockSpec((_QB, 2), lambda i: (i, 0)),
            pl.BlockSpec((_QB, 32), lambda i: (i, 0)),
            pl.BlockSpec((_QB, 32), lambda i: (i, 0)),
            full((2, 16)),
            full((32, 32)),
            full((1, 32)),
            full((_IN_C * 40, _OUT_C)),
            full((1, _OUT_C)),
            full((_OUT_C, _OUT_C * 4)),
            full((1, _OUT_C * 4)),
            full((_OUT_C * 4, _OUT_C)),
            full((1, _OUT_C)),
        ],
        out_specs=pl.BlockSpec((_B, _QB, _OUT_C), lambda i: (0, i, 0)),
        out_shape=jax.ShapeDtypeStruct((_B, _NQ, _OUT_C), f32),
    )(xl2, posT, pos8, query_pos, qmw, qmo, B_rff, W_l1.T,
      b_l1.reshape(1, 32), wl2big, bias.reshape(1, _OUT_C),
      W_mlp1.T, b_mlp1.reshape(1, _OUT_C * 4), W_mlp2.T,
      b_mlp2.reshape(1, _OUT_C), )
    return out


def kernel(x, pos, query_pos, qmw, qmo, W_lin, b_lin, W_mlp1, b_mlp1,
           W_mlp2, b_mlp2, B_rff, W_l1, b_l1, W_l2, filt, bias):
    out = _run(x, pos, query_pos, qmw, qmo, W_lin, b_lin, W_mlp1, b_mlp1,
               W_mlp2, b_mlp2, B_rff, W_l1, b_l1, W_l2, filt, bias)
    return (out, query_pos)


# speedup vs baseline: 1.3697x; 1.0002x over previous
import functools

import jax
import jax.numpy as jnp
import numpy as np
from jax.experimental import pallas as pl
from jax.experimental.pallas import tpu as pltpu

_IN_C = 32
_OUT_C = 64
_NQ = 256
_V = 2048
_B = 8
_K = 103
_KP = 104
_QB = 8
_EPS = 1e-8
_SQRT1_2 = 0.7071067811865476


def _gelu(v):
    return 0.5 * v * (1.0 + jax.lax.erf(v * _SQRT1_2))


def _xlin_kernel(x_ref, w_ref, b_ref, o_ref):
    o_ref[...] = (
        jnp.dot(x_ref[...], w_ref[...], preferred_element_type=jnp.float32)
        + b_ref[...]
    )


def _mega_kernel(xl2_ref, posT_ref, pos8_ref, qp_ref, qmw_ref, qmo_ref,
                 brff_ref, wl1_ref, bl1_ref, wl2big_ref, bias_ref,
                 w1_ref, bm1_ref, w2_ref, bm2_ref, out_ref):
    f32 = jnp.float32
    qp = qp_ref[...]
    p0 = posT_ref[0:1, :]
    p1 = posT_ref[1:2, :]
    d0 = qp[:, 0:1] - p0
    d0 = d0 + 0.5
    d0 = d0 - jnp.floor(d0) - 0.5
    d1 = qp[:, 1:2] - p1
    d1 = d1 + 0.5
    d1 = d1 - jnp.floor(d1) - 0.5
    ed = d0 * d0 + d1 * d1

    bits = jax.lax.bitcast_convert_type(ed, jnp.int32)
    prefix = jnp.zeros((_QB, 1), jnp.int32)
    krem = jnp.full((_QB, 1), _K, jnp.int32)
    for bit in range(30, -1, -1):
        high = jax.lax.shift_right_logical(bits, bit + 1)
        bitv = jax.lax.shift_right_logical(bits, bit) & 1
        match0 = jnp.logical_and(high == prefix, bitv == 0)
        cnt0 = jnp.sum(match0.astype(jnp.int32), axis=1, keepdims=True)
        take0 = krem <= cnt0
        prefix = (prefix << 1) | jnp.where(take0, 0, 1)
        krem = jnp.where(take0, krem, krem - cnt0)
    thr_bits = prefix
    sel = bits <= thr_bits
    seli = sel.astype(jnp.int32)
    cs = seli
    sh = 1
    while sh < _V:
        cs = cs + jnp.concatenate(
            [jnp.zeros((_QB, sh), jnp.int32), cs[:, :-sh]], axis=1)
        sh *= 2
    sel2 = jnp.logical_and(sel, cs <= _K)
    posn = cs - 1

    thr_val = jax.lax.bitcast_convert_type(thr_bits, f32)
    mn = jnp.min(ed, axis=1, keepdims=True)
    en = (ed - mn) / (thr_val - mn + _EPS)
    w = jnp.exp(-en) * sel2.astype(f32)
    kd = w / jnp.sum(w, axis=1, keepdims=True)

    iota = jax.lax.broadcasted_iota(jnp.int32, (_KP, 1), 0)
    xl2 = xl2_ref[...]
    wl2big = wl2big_ref[...]

    ys = []
    for i in range(_QB):
        onehot = jnp.logical_and(posn[i:i + 1, :] == iota,
                                 sel2[i:i + 1, :]).astype(f32)
        at_kd = onehot * kd[i:i + 1, :]
        gxk = jax.lax.dot_general(
            at_kd, xl2, (((1,), (0,)), ((), ())),
            preferred_element_type=f32)
        pg = jnp.dot(onehot, pos8_ref[...],
                     preferred_element_type=f32)
        dg0 = qp[i:i + 1, 0:1] - pg[:, 0:1]
        dg0 = dg0 + 0.5
        dg0 = dg0 - jnp.floor(dg0) - 0.5
        dg1 = qp[i:i + 1, 1:2] - pg[:, 1:2]
        dg1 = dg1 + 0.5
        dg1 = dg1 - jnp.floor(dg1) - 0.5
        proj = (2.0 * np.pi) * (dg0 * brff_ref[0:1, :] + dg1 * brff_ref[1:2, :])
        kf = jnp.concatenate([jnp.sin(proj), jnp.cos(proj)], axis=1)
        h = jnp.dot(kf, wl1_ref[...], preferred_element_type=f32) + bl1_ref[...]
        h = h * qmw_ref[i:i + 1, :] + qmo_ref[i:i + 1, :]
        h = _gelu(h)
        ha = jnp.concatenate(
            [h, jnp.ones((_KP, 1), f32), jnp.zeros((_KP, 7), f32)], axis=1)
        mt = jax.lax.dot_general(
            gxk, ha, (((0,), (0,)), ((), ())),
            preferred_element_type=f32)
        a = jnp.concatenate(
            [mt[c * 8:(c + 1) * 8, :] for c in range(_IN_C)],
            axis=1)
        ys.append(jnp.dot(a, wl2big, preferred_element_type=f32))
    y = jnp.stack(ys, axis=1)
    z = _gelu(y + bias_ref[...].reshape(1, 1, _OUT_C))
    zr = z.reshape(_B * _QB, _OUT_C)
    m = _gelu(jnp.dot(zr, w1_ref[...], preferred_element_type=f32)
              + bm1_ref[...])
    m = jnp.dot(m, w2_ref[...], preferred_element_type=f32) + bm2_ref[...]
    out_ref[...] = (m + zr).reshape(_B, _QB, _OUT_C)


@functools.partial(jax.jit, static_argnames=())
def _run(x, pos, query_pos, qmw, qmo, W_lin, b_lin, W_mlp1, b_mlp1,
         W_mlp2, b_mlp2, B_rff, W_l1, b_l1, W_l2, filt, bias):
    f32 = jnp.float32
    xf = x.reshape(_B * _V, _IN_C)
    xl = pl.pallas_call(
        _xlin_kernel,
        out_shape=jax.ShapeDtypeStruct((_B * _V, _IN_C), f32),
    )(xf, W_lin.T, b_lin.reshape(1, _IN_C))
    xl2 = xl.reshape(_B, _V, _IN_C).transpose(1, 2, 0).reshape(_V, _IN_C * _B)
    posT = pos.T
    pos8 = jnp.concatenate([pos, jnp.zeros((_V, 6), f32)], axis=1)
    wa = jnp.concatenate(
        [W_l2.T, filt.reshape(1, _V), jnp.zeros((7, _V), f32)], axis=0)
    wl2big = wa.reshape(40, _IN_C, _OUT_C).transpose(1, 0, 2).reshape(
        _IN_C * 40, _OUT_C)

    grid = (_NQ // _QB,)
    full = lambda shape: pl.BlockSpec(shape, lambda i: tuple(0 for _ in shape))
    out = pl.pallas_call(
        _mega_kernel,
        grid=grid,
        in_specs=[
            full((_V, _IN_C * _B)),
            full((2, _V)),
            full((_V, 8)),
            pl.BlockSpec((_QB, 2), lambda i: (i, 0)),
            pl.BlockSpec((_QB, 32), lambda i: (i, 0)),
            pl.BlockSpec((_QB, 32), lambda i: (i, 0)),
            full((2, 16)),
            full((32, 32)),
            full((1, 32)),
            full((_IN_C * 40, _OUT_C)),
            full((1, _OUT_C)),
            full((_OUT_C, _OUT_C * 4)),
            full((1, _OUT_C * 4)),
            full((_OUT_C * 4, _OUT_C)),
            full((1, _OUT_C)),
        ],
        out_specs=pl.BlockSpec((_B, _QB, _OUT_C), lambda i: (0, i, 0)),
        out_shape=jax.ShapeDtypeStruct((_B, _NQ, _OUT_C), f32),
        compiler_params=pltpu.CompilerParams(
            dimension_semantics=("parallel",)),
    )(xl2, posT, pos8, query_pos, qmw, qmo, B_rff, W_l1.T,
      b_l1.reshape(1, 32), wl2big, bias.reshape(1, _OUT_C),
      W_mlp1.T, b_mlp1.reshape(1, _OUT_C * 4), W_mlp2.T,
      b_mlp2.reshape(1, _OUT_C), )
    return out


def kernel(x, pos, query_pos, qmw, qmo, W_lin, b_lin, W_mlp1, b_mlp1,
           W_mlp2, b_mlp2, B_rff, W_l1, b_l1, W_l2, filt, bias):
    out = _run(x, pos, query_pos, qmw, qmo, W_lin, b_lin, W_mlp1, b_mlp1,
               W_mlp2, b_mlp2, B_rff, W_l1, b_l1, W_l2, filt, bias)
    return (out, query_pos)


# speedup vs baseline: 1.6533x; 1.2071x over previous
import functools

import jax
import jax.numpy as jnp
import numpy as np
from jax.experimental import pallas as pl
from jax.experimental.pallas import tpu as pltpu

_IN_C = 32
_OUT_C = 64
_NQ = 256
_V = 2048
_B = 8
_K = 103
_KP = 104
_QB = 8
_EPS = 1e-8
_SQRT1_2 = 0.7071067811865476


def _gelu(v):
    return 0.5 * v * (1.0 + jax.lax.erf(v * _SQRT1_2))


def _xlin_kernel(x_ref, w_ref, b_ref, o_ref):
    o_ref[...] = (
        jnp.dot(x_ref[...], w_ref[...], preferred_element_type=jnp.float32)
        + b_ref[...]
    )


def _mega_kernel(xbig_ref, posT_ref, qp_ref, qmw_ref, qmo_ref,
                 brff_ref, wl1_ref, bl1_ref, wl2big_ref, bias_ref,
                 w1_ref, bm1_ref, w2_ref, bm2_ref, out_ref):
    f32 = jnp.float32
    qp = qp_ref[...]
    p0 = posT_ref[0:1, :]
    p1 = posT_ref[1:2, :]
    d0 = qp[:, 0:1] - p0
    d0 = d0 + 0.5
    d0 = d0 - jnp.floor(d0) - 0.5
    d1 = qp[:, 1:2] - p1
    d1 = d1 + 0.5
    d1 = d1 - jnp.floor(d1) - 0.5
    ed = d0 * d0 + d1 * d1

    bits = jax.lax.bitcast_convert_type(ed, jnp.int32)
    prefix = jnp.zeros((_QB, 1), jnp.int32)
    krem = jnp.full((_QB, 1), _K, jnp.int32)
    for bit in range(30, -1, -1):
        high = jax.lax.shift_right_logical(bits, bit + 1)
        bitv = jax.lax.shift_right_logical(bits, bit) & 1
        match0 = jnp.logical_and(high == prefix, bitv == 0)
        cnt0 = jnp.sum(match0.astype(jnp.int32), axis=1, keepdims=True)
        take0 = krem <= cnt0
        prefix = (prefix << 1) | jnp.where(take0, 0, 1)
        krem = jnp.where(take0, krem, krem - cnt0)
    thr_bits = prefix
    sel = bits <= thr_bits
    seli = sel.astype(jnp.int32)
    cs = seli
    sh = 1
    while sh < _V:
        cs = cs + jnp.concatenate(
            [jnp.zeros((_QB, sh), jnp.int32), cs[:, :-sh]], axis=1)
        sh *= 2
    sel2 = jnp.logical_and(sel, cs <= _K)
    posn = cs - 1

    thr_val = jax.lax.bitcast_convert_type(thr_bits, f32)
    mn = jnp.min(ed, axis=1, keepdims=True)
    en = (ed - mn) / (thr_val - mn + _EPS)
    w = jnp.exp(-en) * sel2.astype(f32)
    kd = w / jnp.sum(w, axis=1, keepdims=True)

    iota = jax.lax.broadcasted_iota(jnp.int32, (_KP, 1), 0)
    xbig = xbig_ref[...]
    wl2big = wl2big_ref[...]

    _R = _QB * _KP
    gxs = []
    for i in range(_QB):
        onehot = jnp.logical_and(posn[i:i + 1, :] == iota,
                                 sel2[i:i + 1, :]).astype(f32)
        at_kd = onehot * kd[i:i + 1, :]
        gxs.append(jax.lax.dot_general(
            at_kd, xbig, (((1,), (0,)), ((), ())),
            preferred_element_type=f32))
    g = jnp.concatenate(gxs, axis=0)

    kdg = jnp.maximum(g[:, 258:259], 1e-30)
    pg0 = g[:, 256:257] / kdg
    pg1 = g[:, 257:258] / kdg
    ri = jax.lax.broadcasted_iota(jnp.int32, (_R, 1), 0) // _KP
    qx = jnp.zeros((_R, 1), f32)
    qy = jnp.zeros((_R, 1), f32)
    qmwr = jnp.zeros((_R, 32), f32)
    qmor = jnp.zeros((_R, 32), f32)
    for i in range(_QB):
        hit = ri == i
        qx = jnp.where(hit, qp[i:i + 1, 0:1], qx)
        qy = jnp.where(hit, qp[i:i + 1, 1:2], qy)
        qmwr = jnp.where(hit, qmw_ref[i:i + 1, :], qmwr)
        qmor = jnp.where(hit, qmo_ref[i:i + 1, :], qmor)
    dg0 = qx - pg0
    dg0 = dg0 + 0.5
    dg0 = dg0 - jnp.floor(dg0) - 0.5
    dg1 = qy - pg1
    dg1 = dg1 + 0.5
    dg1 = dg1 - jnp.floor(dg1) - 0.5
    proj = (2.0 * np.pi) * (dg0 * brff_ref[0:1, :] + dg1 * brff_ref[1:2, :])
    wl1 = wl1_ref[...]
    h = (jnp.dot(jnp.sin(proj), wl1[0:16, :], preferred_element_type=f32)
         + jnp.dot(jnp.cos(proj), wl1[16:32, :], preferred_element_type=f32)
         + bl1_ref[...])
    h = _gelu(h * qmwr + qmor)
    ha = jnp.concatenate(
        [h, jnp.ones((_R, 1), f32), jnp.zeros((_R, 7), f32)], axis=1)

    ys = []
    for i in range(_QB):
        sl = slice(i * _KP, (i + 1) * _KP)
        mt = jax.lax.dot_general(
            g[sl, 0:_IN_C * _B], ha[sl, :], (((0,), (0,)), ((), ())),
            preferred_element_type=f32)
        a = jnp.concatenate(
            [mt[c * 8:(c + 1) * 8, :] for c in range(_IN_C)],
            axis=1)
        ys.append(jnp.dot(a, wl2big, preferred_element_type=f32))
    y = jnp.stack(ys, axis=1)
    z = _gelu(y + bias_ref[...].reshape(1, 1, _OUT_C))
    zr = z.reshape(_B * _QB, _OUT_C)
    m = _gelu(jnp.dot(zr, w1_ref[...], preferred_element_type=f32)
              + bm1_ref[...])
    m = jnp.dot(m, w2_ref[...], preferred_element_type=f32) + bm2_ref[...]
    out_ref[...] = (m + zr).reshape(_B, _QB, _OUT_C)


@functools.partial(jax.jit, static_argnames=())
def _run(x, pos, query_pos, qmw, qmo, W_lin, b_lin, W_mlp1, b_mlp1,
         W_mlp2, b_mlp2, B_rff, W_l1, b_l1, W_l2, filt, bias):
    f32 = jnp.float32
    xf = x.reshape(_B * _V, _IN_C)
    xl = pl.pallas_call(
        _xlin_kernel,
        out_shape=jax.ShapeDtypeStruct((_B * _V, _IN_C), f32),
    )(xf, W_lin.T, b_lin.reshape(1, _IN_C))
    xl2 = xl.reshape(_B, _V, _IN_C).transpose(1, 2, 0).reshape(_V, _IN_C * _B)
    posT = pos.T
    xbig = jnp.concatenate(
        [xl2, pos, jnp.ones((_V, 1), f32), jnp.zeros((_V, 13), f32)], axis=1)
    wa = jnp.concatenate(
        [W_l2.T, filt.reshape(1, _V), jnp.zeros((7, _V), f32)], axis=0)
    wl2big = wa.reshape(40, _IN_C, _OUT_C).transpose(1, 0, 2).reshape(
        _IN_C * 40, _OUT_C)

    grid = (_NQ // _QB,)
    full = lambda shape: pl.BlockSpec(shape, lambda i: tuple(0 for _ in shape))
    out = pl.pallas_call(
        _mega_kernel,
        grid=grid,
        in_specs=[
            full((_V, 272)),
            full((2, _V)),
            pl.BlockSpec((_QB, 2), lambda i: (i, 0)),
            pl.BlockSpec((_QB, 32), lambda i: (i, 0)),
            pl.BlockSpec((_QB, 32), lambda i: (i, 0)),
            full((2, 16)),
            full((32, 32)),
            full((1, 32)),
            full((_IN_C * 40, _OUT_C)),
            full((1, _OUT_C)),
            full((_OUT_C, _OUT_C * 4)),
            full((1, _OUT_C * 4)),
            full((_OUT_C * 4, _OUT_C)),
            full((1, _OUT_C)),
        ],
        out_specs=pl.BlockSpec((_B, _QB, _OUT_C), lambda i: (0, i, 0)),
        out_shape=jax.ShapeDtypeStruct((_B, _NQ, _OUT_C), f32),
        compiler_params=pltpu.CompilerParams(
            dimension_semantics=("parallel",)),
    )(xbig, posT, query_pos, qmw, qmo, B_rff, W_l1.T,
      b_l1.reshape(1, 32), wl2big, bias.reshape(1, _OUT_C),
      W_mlp1.T, b_mlp1.reshape(1, _OUT_C * 4), W_mlp2.T,
      b_mlp2.reshape(1, _OUT_C), )
    return out


def kernel(x, pos, query_pos, qmw, qmo, W_lin, b_lin, W_mlp1, b_mlp1,
           W_mlp2, b_mlp2, B_rff, W_l1, b_l1, W_l2, filt, bias):
    out = _run(x, pos, query_pos, qmw, qmo, W_lin, b_lin, W_mlp1, b_mlp1,
               W_mlp2, b_mlp2, B_rff, W_l1, b_l1, W_l2, filt, bias)
    return (out, query_pos)


# speedup vs baseline: 1.7175x; 1.0388x over previous
import functools

import jax
import jax.numpy as jnp
import numpy as np
from jax.experimental import pallas as pl
from jax.experimental.pallas import tpu as pltpu

_IN_C = 32
_OUT_C = 64
_NQ = 256
_V = 2048
_B = 8
_K = 103
_KP = 104
_QB = 8
_EPS = 1e-8
_SQRT1_2 = 0.7071067811865476


def _gelu(v):
    return 0.5 * v * (1.0 + jax.lax.erf(v * _SQRT1_2))


def _xlin_kernel(x_ref, w_ref, b_ref, o_ref):
    o_ref[...] = (
        jnp.dot(x_ref[...], w_ref[...], preferred_element_type=jnp.float32)
        + b_ref[...]
    )


def _mega_kernel(xbig_ref, posT_ref, qp_ref, qmw_ref, qmo_ref,
                 brff2_ref, shift2_ref, wl1_ref, bl1_ref, wl2big_ref,
                 bias_ref, w1_ref, bm1_ref, w2_ref, bm2_ref, out_ref):
    f32 = jnp.float32
    qp = qp_ref[...]
    p0 = posT_ref[0:1, :]
    p1 = posT_ref[1:2, :]
    d0 = qp[:, 0:1] - p0
    d0 = d0 + 0.5
    d0 = d0 - jnp.floor(d0) - 0.5
    d1 = qp[:, 1:2] - p1
    d1 = d1 + 0.5
    d1 = d1 - jnp.floor(d1) - 0.5
    ed = d0 * d0 + d1 * d1

    bits = jax.lax.bitcast_convert_type(ed, jnp.int32)
    prefix = jnp.zeros((_QB, 1), jnp.int32)
    krem = jnp.full((_QB, 1), _K, jnp.int32)
    for bit in range(30, -1, -1):
        high = jax.lax.shift_right_logical(bits, bit + 1)
        bitv = jax.lax.shift_right_logical(bits, bit) & 1
        match0 = jnp.logical_and(high == prefix, bitv == 0)
        cnt0 = jnp.sum(match0.astype(jnp.int32), axis=1, keepdims=True)
        take0 = krem <= cnt0
        prefix = (prefix << 1) | jnp.where(take0, 0, 1)
        krem = jnp.where(take0, krem, krem - cnt0)
    thr_bits = prefix
    sel = bits <= thr_bits
    seli = sel.astype(jnp.int32)
    cs = seli
    sh = 1
    while sh < _V:
        cs = cs + jnp.concatenate(
            [jnp.zeros((_QB, sh), jnp.int32), cs[:, :-sh]], axis=1)
        sh *= 2
    sel2 = jnp.logical_and(sel, cs <= _K)
    posn = cs - 1

    thr_val = jax.lax.bitcast_convert_type(thr_bits, f32)
    mn = jnp.min(ed, axis=1, keepdims=True)
    en = (ed - mn) / (thr_val - mn + _EPS)
    w = jnp.exp(-en) * sel2.astype(f32)
    kd = w / jnp.sum(w, axis=1, keepdims=True)

    iota = jax.lax.broadcasted_iota(jnp.int32, (_KP, 1), 0)
    xbig = xbig_ref[...]
    wl2big = wl2big_ref[...]

    _R = _QB * _KP
    gxs = []
    for i in range(_QB):
        onehot = jnp.logical_and(posn[i:i + 1, :] == iota,
                                 sel2[i:i + 1, :]).astype(f32)
        at_kd = onehot * kd[i:i + 1, :]
        gxs.append(jax.lax.dot_general(
            at_kd, xbig, (((1,), (0,)), ((), ())),
            preferred_element_type=f32))
    g = jnp.concatenate(gxs, axis=0)

    kdg = jnp.maximum(g[:, 258:259], 1e-30)
    pg0 = g[:, 256:257] / kdg
    pg1 = g[:, 257:258] / kdg
    ri = jax.lax.broadcasted_iota(jnp.int32, (_R, 1), 0) // _KP
    qx = jnp.zeros((_R, 1), f32)
    qy = jnp.zeros((_R, 1), f32)
    qmwr = jnp.zeros((_R, 32), f32)
    qmor = jnp.zeros((_R, 32), f32)
    for i in range(_QB):
        hit = ri == i
        qx = jnp.where(hit, qp[i:i + 1, 0:1], qx)
        qy = jnp.where(hit, qp[i:i + 1, 1:2], qy)
        qmwr = jnp.where(hit, qmw_ref[i:i + 1, :], qmwr)
        qmor = jnp.where(hit, qmo_ref[i:i + 1, :], qmor)
    dg0 = qx - pg0
    dg0 = dg0 + 0.5
    dg0 = dg0 - jnp.floor(dg0) - 0.5
    dg1 = qy - pg1
    dg1 = dg1 + 0.5
    dg1 = dg1 - jnp.floor(dg1) - 0.5
    proj2 = (dg0 * brff2_ref[0:1, :] + dg1 * brff2_ref[1:2, :]
             + shift2_ref[...])
    kf = jnp.sin(proj2)
    h = (jnp.dot(kf, wl1_ref[...], preferred_element_type=f32)
         + bl1_ref[...])
    h = _gelu(h * qmwr + qmor)
    ha = jnp.concatenate(
        [h, jnp.ones((_R, 1), f32), jnp.zeros((_R, 7), f32)], axis=1)

    ys = []
    for i in range(_QB):
        sl = slice(i * _KP, (i + 1) * _KP)
        mt = jax.lax.dot_general(
            g[sl, 0:_IN_C * _B], ha[sl, :], (((0,), (0,)), ((), ())),
            preferred_element_type=f32)
        a = jnp.concatenate(
            [mt[c * 8:(c + 1) * 8, :] for c in range(_IN_C)],
            axis=1)
        ys.append(jnp.dot(a, wl2big, preferred_element_type=f32))
    y = jnp.stack(ys, axis=1)
    z = _gelu(y + bias_ref[...].reshape(1, 1, _OUT_C))
    zr = z.reshape(_B * _QB, _OUT_C)
    m = _gelu(jnp.dot(zr, w1_ref[...], preferred_element_type=f32)
              + bm1_ref[...])
    m = jnp.dot(m, w2_ref[...], preferred_element_type=f32) + bm2_ref[...]
    out_ref[...] = (m + zr).reshape(_B, _QB, _OUT_C)


@functools.partial(jax.jit, static_argnames=())
def _run(x, pos, query_pos, qmw, qmo, W_lin, b_lin, W_mlp1, b_mlp1,
         W_mlp2, b_mlp2, B_rff, W_l1, b_l1, W_l2, filt, bias):
    f32 = jnp.float32
    xf = x.reshape(_B * _V, _IN_C)
    xl = pl.pallas_call(
        _xlin_kernel,
        out_shape=jax.ShapeDtypeStruct((_B * _V, _IN_C), f32),
    )(xf, W_lin.T, b_lin.reshape(1, _IN_C))
    xl2 = xl.reshape(_B, _V, _IN_C).transpose(1, 2, 0).reshape(_V, _IN_C * _B)
    posT = pos.T
    xbig = jnp.concatenate(
        [xl2, pos, jnp.ones((_V, 1), f32), jnp.zeros((_V, 13), f32)], axis=1)
    wa = jnp.concatenate(
        [W_l2.T, filt.reshape(1, _V), jnp.zeros((7, _V), f32)], axis=0)
    wl2big = wa.reshape(40, _IN_C, _OUT_C).transpose(1, 0, 2).reshape(
        _IN_C * 40, _OUT_C)
    brff2 = jnp.tile((2.0 * np.pi) * B_rff, (1, 2))
    shift2 = jnp.concatenate(
        [jnp.zeros((1, 16), f32),
         jnp.full((1, 16), 0.5 * np.pi, f32)], axis=1)

    grid = (_NQ // _QB,)
    full = lambda shape: pl.BlockSpec(shape, lambda i: tuple(0 for _ in shape))
    out = pl.pallas_call(
        _mega_kernel,
        grid=grid,
        in_specs=[
            full((_V, 272)),
            full((2, _V)),
            pl.BlockSpec((_QB, 2), lambda i: (i, 0)),
            pl.BlockSpec((_QB, 32), lambda i: (i, 0)),
            pl.BlockSpec((_QB, 32), lambda i: (i, 0)),
            full((2, 32)),
            full((1, 32)),
            full((32, 32)),
            full((1, 32)),
            full((_IN_C * 40, _OUT_C)),
            full((1, _OUT_C)),
            full((_OUT_C, _OUT_C * 4)),
            full((1, _OUT_C * 4)),
            full((_OUT_C * 4, _OUT_C)),
            full((1, _OUT_C)),
        ],
        out_specs=pl.BlockSpec((_B, _QB, _OUT_C), lambda i: (0, i, 0)),
        out_shape=jax.ShapeDtypeStruct((_B, _NQ, _OUT_C), f32),
        compiler_params=pltpu.CompilerParams(
            dimension_semantics=("parallel",)),
    )(xbig, posT, query_pos, qmw, qmo, brff2, shift2, W_l1.T,
      b_l1.reshape(1, 32), wl2big, bias.reshape(1, _OUT_C),
      W_mlp1.T, b_mlp1.reshape(1, _OUT_C * 4), W_mlp2.T,
      b_mlp2.reshape(1, _OUT_C), )
    return out


def kernel(x, pos, query_pos, qmw, qmo, W_lin, b_lin, W_mlp1, b_mlp1,
           W_mlp2, b_mlp2, B_rff, W_l1, b_l1, W_l2, filt, bias):
    out = _run(x, pos, query_pos, qmw, qmo, W_lin, b_lin, W_mlp1, b_mlp1,
               W_mlp2, b_mlp2, B_rff, W_l1, b_l1, W_l2, filt, bias)
    return (out, query_pos)
